# baseline (speedup 1.0000x reference)
import functools

import jax
import jax.numpy as jnp
from jax import lax
from jax.experimental import pallas as pl
from jax.experimental.pallas import tpu as pltpu
from jax.experimental.pallas import tpu_sc as plsc

EMBED_DIM = 64


@functools.lru_cache(maxsize=None)
def _make_gather(B: int):
    D = EMBED_DIM
    info = plsc.get_sparse_core_info()
    NC, NS, L = info.num_cores, info.num_subcores, info.num_lanes
    NW = NC * NS
    assert B % NW == 0
    b_per_w = B // NW
    CH = 320
    n_ch = b_per_w // CH
    assert n_ch * CH == b_per_w and CH % L == 0

    mesh = plsc.VectorSubcoreMesh(core_axis_name="c", subcore_axis_name="s")

    @functools.partial(
        pl.kernel,
        mesh=mesh,
        out_type=jax.ShapeDtypeStruct((B, 2 * D), jnp.float32),
        scratch_types=[
            pltpu.VMEM((b_per_w,), jnp.int32),
            pltpu.VMEM((CH, 2 * D), jnp.float32),
            pltpu.SemaphoreType.DMA,
        ],
        compiler_params=pltpu.CompilerParams(needs_layout_passes=False),
    )
    def gather_kernel(table128_hbm, idx_hbm, out_hbm, idx_v, pairs_v, sem):
        wid = lax.axis_index("s") * NC + lax.axis_index("c")
        base = wid * b_per_w
        pltpu.sync_copy(idx_hbm.at[pl.ds(base, b_per_w)], idx_v)

        def chunk(i, _):
            pltpu.async_copy(
                table128_hbm.at[idx_v.at[pl.ds(i * CH, CH)]], pairs_v, sem
            ).wait()
            pltpu.sync_copy(pairs_v, out_hbm.at[pl.ds(base + i * CH, CH)])
            return _
        lax.fori_loop(0, n_ch, chunk, None)

    return gather_kernel


def kernel(table, inputs):
    shape = inputs.shape
    idx = inputs.reshape(-1).astype(jnp.int32)
    table128 = jnp.pad(table, ((0, 0), (0, table.shape[1])))
    out = _make_gather(idx.shape[0])(table128, idx)
    return out[:, : table.shape[1]].reshape(*shape, table.shape[1])
